# baseline (speedup 1.0000x reference)
import functools

import jax
import jax.numpy as jnp
from jax import lax
from jax.experimental import pallas as pl
from jax.experimental.pallas import tpu as pltpu
from jax.experimental.pallas import tpu_sc as plsc

_NBUF = 8
_CHUNK = 64


def _gather_kernel(N, D, NW, per_w, C, nch):
    mesh = plsc.VectorSubcoreMesh(core_axis_name="c", subcore_axis_name="s")
    NBUF = _NBUF
    ngroups = nch // NBUF

    sem_types = [pltpu.SemaphoreType.DMA] * (2 * NBUF)

    @functools.partial(
        pl.kernel,
        mesh=mesh,
        out_type=jax.ShapeDtypeStruct((N, D), jnp.float32),
        scratch_types=[
            pltpu.VMEM((nch, C), jnp.int32),
            pltpu.VMEM((NBUF, C, D), jnp.float32),
        ]
        + sem_types,
    )
    def k(idx_hbm, table_hbm, out_hbm, idx_v, rows_v,
          g0, g1, g2, g3, g4, g5, g6, g7,
          s0, s1, s2, s3, s4, s5, s6, s7):
        gsem = (g0, g1, g2, g3, g4, g5, g6, g7)
        ssem = (s0, s1, s2, s3, s4, s5, s6, s7)
        wid = lax.axis_index("s") * 2 + lax.axis_index("c")
        base = wid * per_w
        pltpu.sync_copy(idx_hbm.at[wid], idx_v)

        for b in range(NBUF):
            pltpu.async_copy(table_hbm.at[idx_v.at[b]], rows_v.at[b], gsem[b])

        def body(t, _):
            c0 = t * NBUF
            for b in range(NBUF):
                pltpu.make_async_copy(
                    table_hbm.at[idx_v.at[0]], rows_v.at[b], gsem[b]).wait()
                pltpu.async_copy(
                    rows_v.at[b], out_hbm.at[pl.ds(base + (c0 + b) * C, C)],
                    ssem[b])
            for b in range(NBUF):
                pltpu.make_async_copy(
                    rows_v.at[b], out_hbm.at[pl.ds(base, C)], ssem[b]).wait()

                @pl.when(t < ngroups - 1)
                def _issue(b=b, c0=c0):
                    pltpu.async_copy(
                        table_hbm.at[idx_v.at[c0 + NBUF + b]], rows_v.at[b],
                        gsem[b])

            return 0

        lax.fori_loop(0, ngroups, body, 0)

    return k


def kernel(categories, ce):
    B, S = categories.shape
    V, D = ce.shape
    N = B * S
    NW = 32
    per_w = N // NW
    C = _CHUNK
    nch = per_w // C
    idx3 = categories.reshape(NW, nch, C)
    out = _gather_kernel(N, D, NW, per_w, C, nch)(idx3, ce)
    return out.reshape(B, S, D)


# speedup vs baseline: 4.8471x; 4.8471x over previous
import functools

import jax
import jax.numpy as jnp
from jax import lax
from jax.experimental import pallas as pl
from jax.experimental.pallas import tpu as pltpu
from jax.experimental.pallas import tpu_sc as plsc

_NBUF = 8
_CHUNK = 64


def _gather_kernel(N, D, V, NW, per_w, C, nch):
    mesh = plsc.VectorSubcoreMesh(core_axis_name="c", subcore_axis_name="s")
    NBUF = _NBUF
    ngroups = nch // NBUF

    sem_types = [pltpu.SemaphoreType.DMA] * (2 * NBUF)

    @functools.partial(
        pl.kernel,
        mesh=mesh,
        out_type=jax.ShapeDtypeStruct((N, D), jnp.float32),
        scratch_types=[
            pltpu.VMEM((nch, C), jnp.int32),
            pltpu.VMEM_SHARED((V, D), jnp.float32),
            pltpu.VMEM((NBUF, C, D), jnp.float32),
        ]
        + sem_types,
    )
    def k(idx_hbm, table_hbm, out_hbm, idx_v, table_sh, rows_v,
          g0, g1, g2, g3, g4, g5, g6, g7,
          s0, s1, s2, s3, s4, s5, s6, s7):
        gsem = (g0, g1, g2, g3, g4, g5, g6, g7)
        ssem = (s0, s1, s2, s3, s4, s5, s6, s7)
        sid = lax.axis_index("s")
        wid = sid * 2 + lax.axis_index("c")
        base = wid * per_w

        @pl.when(sid == 0)
        def _stage_table():
            pltpu.sync_copy(table_hbm, table_sh)

        pltpu.sync_copy(idx_hbm.at[wid], idx_v)
        plsc.subcore_barrier()

        for b in range(NBUF):
            pltpu.async_copy(table_sh.at[idx_v.at[b]], rows_v.at[b], gsem[b])

        def body(t, _):
            c0 = t * NBUF
            for b in range(NBUF):
                pltpu.make_async_copy(
                    table_sh.at[idx_v.at[0]], rows_v.at[b], gsem[b]).wait()
                pltpu.async_copy(
                    rows_v.at[b], out_hbm.at[pl.ds(base + (c0 + b) * C, C)],
                    ssem[b])
            for b in range(NBUF):
                pltpu.make_async_copy(
                    rows_v.at[b], out_hbm.at[pl.ds(base, C)], ssem[b]).wait()

                @pl.when(t < ngroups - 1)
                def _issue(b=b, c0=c0):
                    pltpu.async_copy(
                        table_sh.at[idx_v.at[c0 + NBUF + b]], rows_v.at[b],
                        gsem[b])

            return 0

        lax.fori_loop(0, ngroups, body, 0)

    return k


def kernel(categories, ce):
    B, S = categories.shape
    V, D = ce.shape
    N = B * S
    NW = 32
    per_w = N // NW
    C = _CHUNK
    nch = per_w // C
    idx3 = categories.reshape(NW, nch, C)
    out = _gather_kernel(N, D, V, NW, per_w, C, nch)(idx3, ce)
    return out.reshape(B, S, D)
